# baseline (speedup 1.0000x reference)
import jax
import jax.numpy as jnp
from jax.experimental import pallas as pl

_TILE = 3584


def _proj_kernel(x_ref, rm_ref, o_ref):
    o_ref[0] = jax.lax.dot_general(
        x_ref[0],
        rm_ref[0],
        dimension_numbers=(((0,), (0,)), ((), ())),
        preferred_element_type=jnp.float32,
    )


def kernel(inputs, random_matrices):
    n, c, h, w = inputs.shape
    hw = h * w
    m = random_matrices.shape[2]
    x = inputs.reshape(n, c, hw)

    tile = _TILE if hw % _TILE == 0 else hw
    grid = (n, hw // tile)

    return pl.pallas_call(
        _proj_kernel,
        grid=grid,
        in_specs=[
            pl.BlockSpec((1, c, tile), lambda b, t: (b, 0, t)),
            pl.BlockSpec((1, c, m), lambda b, t: (b, 0, 0)),
        ],
        out_specs=pl.BlockSpec((1, tile, m), lambda b, t: (b, t, 0)),
        out_shape=jax.ShapeDtypeStruct((n, hw, m), jnp.float32),
    )(x, random_matrices)


# speedup vs baseline: 1.0154x; 1.0154x over previous
import jax
import jax.numpy as jnp
from jax.experimental import pallas as pl

_TILE = 3584


def _proj_kernel(x_ref, rm_ref, o_ref):
    o_ref[0] = jax.lax.dot_general(
        x_ref[0].astype(jnp.bfloat16),
        rm_ref[0].astype(jnp.bfloat16),
        dimension_numbers=(((0,), (0,)), ((), ())),
        preferred_element_type=jnp.float32,
    )


def kernel(inputs, random_matrices):
    n, c, h, w = inputs.shape
    hw = h * w
    m = random_matrices.shape[2]
    x = inputs.reshape(n, c, hw)

    tile = _TILE if hw % _TILE == 0 else hw
    grid = (n, hw // tile)

    return pl.pallas_call(
        _proj_kernel,
        grid=grid,
        in_specs=[
            pl.BlockSpec((1, c, tile), lambda b, t: (b, 0, t)),
            pl.BlockSpec((1, c, m), lambda b, t: (b, 0, 0)),
        ],
        out_specs=pl.BlockSpec((1, tile, m), lambda b, t: (b, t, 0)),
        out_shape=jax.ShapeDtypeStruct((n, hw, m), jnp.float32),
    )(x, random_matrices)
